# baseline (speedup 1.0000x reference)
import jax
import jax.numpy as jnp
from jax.experimental import pallas as pl
from jax.experimental.pallas import tpu as pltpu

_B, _N, _C, _NCLS = 64, 196, 384, 1000
_K = int((1.0 - 0.7) * _N)
_BB = 8
_G = _B // _BB


def _row_kl_sum_terms(x, xt):
    mx = jnp.max(x, axis=-1, keepdims=True)
    mxt = jnp.max(xt, axis=-1, keepdims=True)
    s_p = jnp.sum(jnp.exp(x - mx), axis=-1)
    w = jnp.exp(xt - mxt)
    s_t = jnp.sum(w, axis=-1)
    dot = jnp.sum(w * (xt - x), axis=-1)
    return dot / s_t - (mxt - mx)[..., 0] + jnp.log(s_p / s_t)


def _body(pred_ref, predt_ref, s0_ref, s1_ref, s2_ref, m0_ref, m1_ref, m2_ref,
          sf_ref, sft_ref, ld_ref, out_ref, acc_ref):
    g = pl.program_id(0)

    @pl.when(g == 0)
    def _init():
        row = _row_kl_sum_terms(pred_ref[...], predt_ref[...])
        acc_ref[0] = jnp.sum(row)
        acc_ref[1] = 0.0
        acc_ref[2] = 0.0
        acc_ref[3] = 0.0
        acc_ref[4] = 0.0

    row_kl = _row_kl_sum_terms(sf_ref[...], sft_ref[...])
    ld = ld_ref[...]
    keep = ld > 0.5
    acc_ref[2] += jnp.sum(jnp.where(keep, row_kl, 0.0))
    acc_ref[3] += jnp.sum(keep.astype(jnp.float32))
    acc_ref[4] += jnp.sum(ld)

    attn_sq = 0.0
    for s_ref, m_ref in ((s0_ref, m0_ref), (s1_ref, m1_ref), (s2_ref, m2_ref)):
        s = s_ref[...][:, :, 1]
        cmp = (s[:, None, :] > s[:, :, None]).astype(jnp.float32)
        rank = jnp.sum(cmp, axis=-1)
        target = (rank >= _K).astype(jnp.float32)
        d = target - m_ref[...]
        attn_sq += jnp.sum(d * d)
    acc_ref[1] += attn_sq

    @pl.when(g == _G - 1)
    def _fin():
        attn = (2.0 / 3.0) * acc_ref[1] / (_B * _N)
        cls_kl = acc_ref[0] / _B
        token = jnp.where(acc_ref[4] < 0.1, 0.0, acc_ref[2] / acc_ref[3])
        total = attn + 0.5 * cls_kl + 0.5 * token
        out_ref[...] = jnp.broadcast_to(total, (1, 1))


def kernel(pred, pred_t, spatial_features, last_decision, spatial_features_t,
           hard_keep_decision_0, hard_keep_decision_1, hard_keep_decision_2,
           token_attn_sim_0, token_attn_sim_1, token_attn_sim_2):
    out = pl.pallas_call(
        _body,
        grid=(_G,),
        in_specs=[
            pl.BlockSpec((_B, _NCLS), lambda g: (0, 0)),
            pl.BlockSpec((_B, _NCLS), lambda g: (0, 0)),
            pl.BlockSpec((_BB, _N, 3), lambda g: (g, 0, 0)),
            pl.BlockSpec((_BB, _N, 3), lambda g: (g, 0, 0)),
            pl.BlockSpec((_BB, _N, 3), lambda g: (g, 0, 0)),
            pl.BlockSpec((_BB, _N), lambda g: (g, 0)),
            pl.BlockSpec((_BB, _N), lambda g: (g, 0)),
            pl.BlockSpec((_BB, _N), lambda g: (g, 0)),
            pl.BlockSpec((_BB, _N, _C), lambda g: (g, 0, 0)),
            pl.BlockSpec((_BB, _N, _C), lambda g: (g, 0, 0)),
            pl.BlockSpec((_BB, _N), lambda g: (g, 0)),
        ],
        out_specs=pl.BlockSpec((1, 1), lambda g: (0, 0)),
        out_shape=jax.ShapeDtypeStruct((1, 1), jnp.float32),
        scratch_shapes=[pltpu.SMEM((8,), jnp.float32)],
    )(pred, pred_t, token_attn_sim_0, token_attn_sim_1, token_attn_sim_2,
      hard_keep_decision_0, hard_keep_decision_1, hard_keep_decision_2,
      spatial_features, spatial_features_t, last_decision)
    return out.reshape(())


# speedup vs baseline: 14.6459x; 14.6459x over previous
import jax
import jax.numpy as jnp
from jax import lax
from jax.experimental import pallas as pl
from jax.experimental.pallas import tpu as pltpu

_B, _N, _C, _NCLS = 64, 196, 384, 1000
_K = int((1.0 - 0.7) * _N)
_BB = 8
_G = _B // _BB
_MININT = -2147483648


def _row_kl_sum_terms(x, xt):
    mx = jnp.max(x, axis=-1, keepdims=True)
    mxt = jnp.max(xt, axis=-1, keepdims=True)
    s_p = jnp.sum(jnp.exp(x - mx), axis=-1)
    w = jnp.exp(xt - mxt)
    s_t = jnp.sum(w, axis=-1)
    dot = jnp.sum(w * (xt - x), axis=-1)
    return dot / s_t - (mxt - mx)[..., 0] + jnp.log(s_p / s_t)


def _sortable_key(x):
    b = lax.bitcast_convert_type(x, jnp.int32)
    sign = lax.shift_right_arithmetic(b, 31)
    return lax.bitwise_xor(b, lax.bitwise_and(sign, jnp.int32(0x7FFFFFFF)))


def _attn_sq_sum(s, m):
    key = _sortable_key(s)
    kth = jnp.full(s.shape[:-1] + (1,), 0, jnp.int32)

    def bit_step(i, prefix_u):
        bit = lax.shift_left(jnp.int32(1), 31 - i)
        cand_u = lax.bitwise_or(prefix_u, bit)
        cand_s = lax.bitwise_xor(cand_u, jnp.int32(_MININT))
        ge = (key >= cand_s).astype(jnp.int32)
        cnt = jnp.sum(ge, axis=-1, keepdims=True)
        return jnp.where(cnt >= _K, cand_u, prefix_u)

    thr_u = lax.fori_loop(0, 32, bit_step, kth)
    thr_s = lax.bitwise_xor(thr_u, jnp.int32(_MININT))
    target = (key < thr_s).astype(jnp.float32)
    d = target - m
    return jnp.sum(d * d)


def _body(pred_ref, predt_ref, s_ref, m_ref, sf_ref, sft_ref, ld_ref,
          out_ref, acc_ref):
    g = pl.program_id(0)

    @pl.when(g == 0)
    def _init():
        row = _row_kl_sum_terms(pred_ref[...], predt_ref[...])
        acc_ref[0] = jnp.sum(row)
        acc_ref[1] = _attn_sq_sum(s_ref[...], m_ref[...])
        acc_ref[2] = 0.0
        acc_ref[3] = 0.0
        acc_ref[4] = 0.0

    row_kl = _row_kl_sum_terms(sf_ref[...], sft_ref[...])
    ld = ld_ref[...]
    keep = ld > 0.5
    acc_ref[2] += jnp.sum(jnp.where(keep, row_kl, 0.0))
    acc_ref[3] += jnp.sum(keep.astype(jnp.float32))
    acc_ref[4] += jnp.sum(ld)

    @pl.when(g == _G - 1)
    def _fin():
        attn = (2.0 / 3.0) * acc_ref[1] / (_B * _N)
        cls_kl = acc_ref[0] / _B
        token = jnp.where(acc_ref[4] < 0.1, 0.0, acc_ref[2] / acc_ref[3])
        total = attn + 0.5 * cls_kl + 0.5 * token
        out_ref[...] = jnp.broadcast_to(total, (1, 1))


def kernel(pred, pred_t, spatial_features, last_decision, spatial_features_t,
           hard_keep_decision_0, hard_keep_decision_1, hard_keep_decision_2,
           token_attn_sim_0, token_attn_sim_1, token_attn_sim_2):
    s_stack = jnp.stack([token_attn_sim_0[:, :, 1],
                         token_attn_sim_1[:, :, 1],
                         token_attn_sim_2[:, :, 1]])
    m_stack = jnp.stack([hard_keep_decision_0, hard_keep_decision_1,
                         hard_keep_decision_2])
    out = pl.pallas_call(
        _body,
        grid=(_G,),
        in_specs=[
            pl.BlockSpec((_B, _NCLS), lambda g: (0, 0)),
            pl.BlockSpec((_B, _NCLS), lambda g: (0, 0)),
            pl.BlockSpec((3, _B, _N), lambda g: (0, 0, 0)),
            pl.BlockSpec((3, _B, _N), lambda g: (0, 0, 0)),
            pl.BlockSpec((_BB, _N, _C), lambda g: (g, 0, 0)),
            pl.BlockSpec((_BB, _N, _C), lambda g: (g, 0, 0)),
            pl.BlockSpec((_BB, _N), lambda g: (g, 0)),
        ],
        out_specs=pl.BlockSpec((1, 1), lambda g: (0, 0)),
        out_shape=jax.ShapeDtypeStruct((1, 1), jnp.float32),
        scratch_shapes=[pltpu.SMEM((8,), jnp.float32)],
    )(pred, pred_t, s_stack, m_stack,
      spatial_features, spatial_features_t, last_decision)
    return out.reshape(())
